# baseline (speedup 1.0000x reference)
import functools

import jax
import jax.numpy as jnp
from jax import lax
from jax.experimental import pallas as pl
from jax.experimental.pallas import tpu as pltpu
from jax.experimental.pallas import tpu_sc as plsc

_FIELD_OFFSET = 1000000
_B = 4096
_D = 16

_NC = 2
_NS = 16
_L = 16
_NW = _NC * _NS
_BPW = _B // _NW


def _mf_body(x_hbm, table_hbm, out_hbm,
             xv, uidx, iidx, urows, irows, outv, sem_u, sem_i):
    wid = lax.axis_index("s") * _NC + lax.axis_index("c")
    base = wid * _BPW

    pltpu.sync_copy(x_hbm.at[pl.ds(2 * base, 2 * _BPW)], xv)

    iota = lax.iota(jnp.int32, _L)
    for i in range(_BPW // _L):
        pos = iota * 2 + (2 * _L) * i
        uv = plsc.load_gather(xv, [pos])
        iv = plsc.load_gather(xv, [pos + 1])
        uidx[pl.ds(i * _L, _L)] = uv
        iidx[pl.ds(i * _L, _L)] = iv + _FIELD_OFFSET

    cu = pltpu.async_copy(table_hbm.at[uidx], urows, sem_u)
    ci = pltpu.async_copy(table_hbm.at[iidx], irows, sem_i)
    cu.wait()
    ci.wait()

    for i in range(_BPW // _L):
        rows = iota + i * _L
        acc = jnp.zeros((_L,), jnp.float32)
        for d in range(_D):
            col = jnp.full((_L,), d, jnp.int32)
            uc = plsc.load_gather(urows, [rows, col])
            ic = plsc.load_gather(irows, [rows, col])
            acc = acc + uc * ic
        outv[pl.ds(i * _L, _L)] = acc

    pltpu.sync_copy(outv, out_hbm.at[pl.ds(base, _BPW)])


@functools.partial(
    pl.kernel,
    out_type=jax.ShapeDtypeStruct((_B,), jnp.float32),
    mesh=plsc.VectorSubcoreMesh(core_axis_name="c", subcore_axis_name="s"),
    compiler_params=pltpu.CompilerParams(
        needs_layout_passes=False, use_tc_tiling_on_sc=False
    ),
    scratch_types=[
        pltpu.VMEM((2 * _BPW,), jnp.int32),
        pltpu.VMEM((_BPW,), jnp.int32),
        pltpu.VMEM((_BPW,), jnp.int32),
        pltpu.VMEM((_BPW, _D), jnp.float32),
        pltpu.VMEM((_BPW, _D), jnp.float32),
        pltpu.VMEM((_BPW,), jnp.float32),
        pltpu.SemaphoreType.DMA,
        pltpu.SemaphoreType.DMA,
    ],
)
def _mf_kernel(x_hbm, table_hbm, out_hbm,
               xv, uidx, iidx, urows, irows, outv, sem_u, sem_i):
    _mf_body(x_hbm, table_hbm, out_hbm,
             xv, uidx, iidx, urows, irows, outv, sem_u, sem_i)


def kernel(x, table):
    y = _mf_kernel(x.reshape(-1), table)
    return y.reshape(_B, 1)


# speedup vs baseline: 32.6336x; 32.6336x over previous
import functools

import jax
import jax.numpy as jnp
from jax import lax
from jax.experimental import pallas as pl
from jax.experimental.pallas import tpu as pltpu
from jax.experimental.pallas import tpu_sc as plsc

_FIELD_OFFSET = 1000000
_B = 4096
_D = 16
_ROWS = 2000000

_NC = 2
_NS = 16
_L = 16
_NW = _NC * _NS
_BPW = _B // _NW

_TILE_R = 128
_TILE_D = 8
_RT = _ROWS // _TILE_R
_DCONST = [(d // _TILE_D) * _RT * 1024 + (d % _TILE_D) * _TILE_R
           for d in range(_D)]


def _mf_body(x_hbm, t_hbm, out_hbm, xv, ub, ib, idxb, gath, outv, sem):
    wid = lax.axis_index("s") * _NC + lax.axis_index("c")
    base = wid * _BPW

    pltpu.sync_copy(x_hbm.at[pl.ds(2 * base, 2 * _BPW)], xv)

    iota = lax.iota(jnp.int32, _L)
    for blk in range(_BPW // _L):
        pos = iota * 2 + (2 * _L) * blk
        u = plsc.load_gather(xv, [pos])
        it = plsc.load_gather(xv, [pos + 1]) + _FIELD_OFFSET
        ub[pl.ds(blk * _L, _L)] = ((u >> 7) << 10) + (u & 127)
        ib[pl.ds(blk * _L, _L)] = ((it >> 7) << 10) + (it & 127)

    for blk in range(_BPW // _L):
        sl = pl.ds(blk * _L, _L)
        uv = ub[sl]
        iv = ib[sl]
        for d in range(_D):
            idxb[d, sl] = uv + _DCONST[d]
            idxb[_D + d, sl] = iv + _DCONST[d]

    copies = [
        pltpu.async_copy(t_hbm.at[idxb.at[k]], gath.at[k], sem)
        for k in range(2 * _D)
    ]
    for c in copies:
        c.wait()

    for blk in range(_BPW // _L):
        sl = pl.ds(blk * _L, _L)
        acc = gath[0, sl] * gath[_D, sl]
        for d in range(1, _D):
            acc = acc + gath[d, sl] * gath[_D + d, sl]
        outv[sl] = acc

    pltpu.sync_copy(outv, out_hbm.at[pl.ds(base, _BPW)])


@functools.partial(
    pl.kernel,
    out_type=jax.ShapeDtypeStruct((_B,), jnp.float32),
    mesh=plsc.VectorSubcoreMesh(core_axis_name="c", subcore_axis_name="s"),
    compiler_params=pltpu.CompilerParams(
        needs_layout_passes=False, use_tc_tiling_on_sc=False
    ),
    scratch_types=[
        pltpu.VMEM((2 * _BPW,), jnp.int32),
        pltpu.VMEM((_BPW,), jnp.int32),
        pltpu.VMEM((_BPW,), jnp.int32),
        pltpu.VMEM((2 * _D, _BPW), jnp.int32),
        pltpu.VMEM((2 * _D, _BPW), jnp.float32),
        pltpu.VMEM((_BPW,), jnp.float32),
        pltpu.SemaphoreType.DMA,
    ],
)
def _mf_kernel(x_hbm, t_hbm, out_hbm, xv, ub, ib, idxb, gath, outv, sem):
    _mf_body(x_hbm, t_hbm, out_hbm, xv, ub, ib, idxb, gath, outv, sem)


def kernel(x, table):
    tflat = (
        table.reshape(_RT, _TILE_R, _D // _TILE_D, _TILE_D)
        .transpose(2, 0, 3, 1)
        .reshape(_ROWS * _D)
    )
    y = _mf_kernel(x.reshape(-1), tflat)
    return y.reshape(_B, 1)
